# baseline (speedup 1.0000x reference)
import functools

import numpy as np

import jax
import jax.numpy as jnp
from jax import lax
from jax.experimental import pallas as pl
from jax.experimental.pallas import tpu as pltpu
from jax.experimental.pallas import tpu_sc as plsc

E_TOTAL = 160000
N_NODES = 10000
E_PAD = 160000
B_TC = 4096
NSLOT = 512

NW = 16
L = 16
CH = E_PAD // NW
VEC = CH // L
NP = 10240
NS = NP // NW

_GAMMA = (0, 2, 0, 2, 1, 3, 1, 3)
_BETA = (0, 0, 2, 2, 1, 3, 3, 1)
_SIGN = (1.0, 1.0, 1.0, 1.0, 1.0, 1.0, 1.0, -1.0)
_WMAP = (0, 1, 2, 3, 4, 4, 5, 5)


def _tc_body(xq_ref, xk_ref, emb_ref, wqe_ref, ek_ref, wp_ref, out_ref):
    q = jnp.dot(xq_ref[...], wqe_ref[...], preferred_element_type=jnp.float32)
    k = jnp.dot(xk_ref[...], ek_ref[...], preferred_element_type=jnp.float32)
    w = jnp.dot(emb_ref[...], wp_ref[...], preferred_element_type=jnp.float32)
    out_ref[...] = jnp.sum(q * k * w, axis=1)


def _tc_pre(xq, xk, emb, wqe, ek, wp, interpret=False):
    grid = ((E_PAD + B_TC - 1) // B_TC,)
    return pl.pallas_call(
        _tc_body,
        grid=grid,
        in_specs=[
            pl.BlockSpec((B_TC, 32), lambda i: (i, 0)),
            pl.BlockSpec((B_TC, 32), lambda i: (i, 0)),
            pl.BlockSpec((B_TC, 16), lambda i: (i, 0)),
            pl.BlockSpec((32, NSLOT), lambda i: (0, 0)),
            pl.BlockSpec((32, NSLOT), lambda i: (0, 0)),
            pl.BlockSpec((16, NSLOT), lambda i: (0, 0)),
        ],
        out_specs=pl.BlockSpec((B_TC,), lambda i: (i,)),
        out_shape=jax.ShapeDtypeStruct((E_PAD,), jnp.float32),
        interpret=interpret,
    )(xq, xk, emb, wqe, ek, wp)


def _sc_softmax_body(pre_hbm, idx_hbm, out_hbm, pre_v, idx_v, e_v, mloc,
                     sloc, glob, comb, tmp, xbuf, ibuf, sh_m, sh_g):
    wid = lax.axis_index("s")
    base = wid * CH
    nbase = wid * NS

    pltpu.sync_copy(pre_hbm.at[pl.ds(base, CH)], pre_v)
    pltpu.sync_copy(idx_hbm.at[pl.ds(base, CH)], idx_v)

    neg = jnp.full((L,), -1e30, jnp.float32)
    zero = jnp.zeros((L,), jnp.float32)

    def init_body(i, c):
        mloc[pl.ds(i * L, L)] = neg
        sloc[pl.ds(i * L, L)] = zero
        return c

    lax.fori_loop(0, NP // L, init_body, 0)

    ibuf[pl.ds(0, L)] = jnp.full((L,), -1, jnp.int32)
    ibuf[pl.ds(2 * L, L)] = jnp.full((L,), -2, jnp.int32)

    xbuf[pl.ds(0, L)] = neg

    def max_step(v, c):
        off = v * L
        x = pre_v[pl.ds(off, L)]
        ids = idx_v[pl.ds(off, L)]
        ibuf[pl.ds(L, L)] = ids
        for d in (1, 2, 4, 8):
            xbuf[pl.ds(L, L)] = x
            y = xbuf[pl.ds(L - d, L)]
            pid = ibuf[pl.ds(L - d, L)]
            x = jnp.where(ids == pid, jnp.maximum(x, y), x)
        nid = ibuf[pl.ds(L + 1, L)]
        endm = ids != nid
        old = plsc.load_gather(mloc, [ids])
        plsc.store_scatter(mloc, [ids], jnp.maximum(x, old), mask=endm)
        return c

    lax.fori_loop(0, VEC, max_step, 0)

    pltpu.sync_copy(mloc, sh_m.at[wid])
    plsc.subcore_barrier()
    for w2 in range(NW):
        pltpu.sync_copy(sh_m.at[w2, pl.ds(nbase, NS)], comb.at[w2])

    def red_max(j, c):
        acc = comb[0, pl.ds(j * L, L)]
        for w2 in range(1, NW):
            acc = jnp.maximum(acc, comb[w2, pl.ds(j * L, L)])
        tmp[pl.ds(j * L, L)] = acc
        return c

    lax.fori_loop(0, NS // L, red_max, 0)
    pltpu.sync_copy(tmp, sh_g.at[pl.ds(nbase, NS)])
    plsc.subcore_barrier()
    pltpu.sync_copy(sh_g, glob)

    xbuf[pl.ds(0, L)] = zero

    def sum_step(v, c):
        off = v * L
        p = pre_v[pl.ds(off, L)]
        ids = idx_v[pl.ds(off, L)]
        z = plsc.load_gather(glob, [ids])
        x = jnp.exp(p - z)
        e_v[pl.ds(off, L)] = x
        ibuf[pl.ds(L, L)] = ids
        for d in (1, 2, 4, 8):
            xbuf[pl.ds(L, L)] = x
            y = xbuf[pl.ds(L - d, L)]
            pid = ibuf[pl.ds(L - d, L)]
            x = jnp.where(ids == pid, x + y, x)
        nid = ibuf[pl.ds(L + 1, L)]
        endm = ids != nid
        old = plsc.load_gather(sloc, [ids])
        plsc.store_scatter(sloc, [ids], x + old, mask=endm)
        return c

    lax.fori_loop(0, VEC, sum_step, 0)

    pltpu.sync_copy(sloc, sh_m.at[wid])
    plsc.subcore_barrier()
    for w2 in range(NW):
        pltpu.sync_copy(sh_m.at[w2, pl.ds(nbase, NS)], comb.at[w2])

    def red_sum(j, c):
        acc = comb[0, pl.ds(j * L, L)]
        for w2 in range(1, NW):
            acc = acc + comb[w2, pl.ds(j * L, L)]
        tmp[pl.ds(j * L, L)] = acc
        return c

    lax.fori_loop(0, NS // L, red_sum, 0)
    pltpu.sync_copy(tmp, sh_g.at[pl.ds(nbase, NS)])
    plsc.subcore_barrier()
    pltpu.sync_copy(sh_g, glob)

    def out_step(v, c):
        off = v * L
        e = e_v[pl.ds(off, L)]
        ids = idx_v[pl.ds(off, L)]
        s = plsc.load_gather(glob, [ids])
        e_v[pl.ds(off, L)] = e / (s + 1e-16)
        return c

    lax.fori_loop(0, VEC, out_step, 0)
    pltpu.sync_copy(e_v, out_hbm.at[pl.ds(base, CH)])


def _sc_softmax(pre, idx):
    mesh = plsc.VectorSubcoreMesh(
        core_axis_name="c", subcore_axis_name="s", num_cores=1,
        num_subcores=NW)
    fn = pl.kernel(
        _sc_softmax_body,
        out_type=jax.ShapeDtypeStruct((E_PAD,), jnp.float32),
        mesh=mesh,
        scratch_types=[
            pltpu.VMEM((CH,), jnp.float32),
            pltpu.VMEM((CH,), jnp.int32),
            pltpu.VMEM((CH,), jnp.float32),
            pltpu.VMEM((NP,), jnp.float32),
            pltpu.VMEM((NP,), jnp.float32),
            pltpu.VMEM((NP,), jnp.float32),
            pltpu.VMEM((NW, NS), jnp.float32),
            pltpu.VMEM((NS,), jnp.float32),
            pltpu.VMEM((2 * L,), jnp.float32),
            pltpu.VMEM((3 * L,), jnp.int32),
            pltpu.VMEM_SHARED((NW, NP), jnp.float32),
            pltpu.VMEM_SHARED((NP,), jnp.float32),
        ],
        compiler_params=pltpu.CompilerParams(needs_layout_passes=False),
    )
    return fn(pre, idx)


def _build_consts():
    m = np.zeros((32 * 32, 384), np.float32)

    def put(rb, cb, w, sign):
        for c in range(8):
            for o in range(8):
                m[(rb * 8 + c) * 32 + cb * 8 + o, w * 64 + c * 8 + o] = sign

    put(0, 0, 0, 1.0)
    put(0, 2, 1, 1.0)
    put(1, 1, 4, 1.0)
    put(1, 3, 5, -1.0)
    put(2, 0, 2, 1.0)
    put(2, 2, 3, 1.0)
    put(3, 1, 5, 1.0)
    put(3, 3, 4, 1.0)

    scale = 8.0 ** -0.5
    cq = np.zeros((32, NSLOT), np.float32)
    ekc = np.zeros((32, NSLOT), np.float32)
    cp = np.zeros((384, NSLOT), np.float32)
    for s in range(8):
        for c in range(8):
            for o in range(8):
                col = s * 64 + c * 8 + o
                cq[_GAMMA[s] * 8 + o, col] = _SIGN[s] * scale
                ekc[_BETA[s] * 8 + c, col] = 1.0
                cp[_WMAP[s] * 64 + c * 8 + o, col] = 1.0
    return m, cq, ekc, cp


_M_B, _C_Q, _EK_CONST, _C_P = _build_consts()


def _prepack(Wq, Wprod):
    wbig = (jnp.asarray(_M_B) @ Wq.reshape(384)).reshape(32, 32)
    wqe = wbig @ jnp.asarray(_C_Q)
    ek = jnp.asarray(_EK_CONST)
    wp = Wprod @ jnp.asarray(_C_P)
    return wqe, ek, wp


def kernel(x_q, x_k, edge_emb, Wq, Wprod, index, num_nodes):
    e = x_q.shape[0]
    xq = x_q.reshape(e, 32)
    xk = x_k.reshape(e, 32)
    idx = jnp.minimum(index, num_nodes - 1).astype(jnp.int32)

    wqe, ek, wp = _prepack(Wq, Wprod)
    pre = _tc_pre(xq, xk, edge_emb, wqe, ek, wp)
    out = _sc_softmax(pre, idx)
    return out.reshape(e, 1)


# speedup vs baseline: 1.6276x; 1.6276x over previous
import functools

import numpy as np

import jax
import jax.numpy as jnp
from jax import lax
from jax.experimental import pallas as pl
from jax.experimental.pallas import tpu as pltpu
from jax.experimental.pallas import tpu_sc as plsc

E_TOTAL = 160000
N_NODES = 10000
E_PAD = 160000
B_TC = 4096
NSLOT = 512

NW = 16
L = 16
CH = E_PAD // NW
VEC = CH // L
NP = 10240
NS = NP // NW

_GAMMA = (0, 2, 0, 2, 1, 3, 1, 3)
_BETA = (0, 0, 2, 2, 1, 3, 3, 1)
_SIGN = (1.0, 1.0, 1.0, 1.0, 1.0, 1.0, 1.0, -1.0)
_WMAP = (0, 1, 2, 3, 4, 4, 5, 5)


def _tc_body(xq_ref, xk_ref, emb_ref, wqe_ref, ek_ref, wp_ref, out_ref):
    q = jnp.dot(xq_ref[...], wqe_ref[...], preferred_element_type=jnp.float32)
    k = jnp.dot(xk_ref[...], ek_ref[...], preferred_element_type=jnp.float32)
    w = jnp.dot(emb_ref[...], wp_ref[...], preferred_element_type=jnp.float32)
    p = (q * k * w).reshape(B_TC // 128, 128, NSLOT)
    out_ref[...] = jnp.sum(p, axis=2)


def _tc_pre(xq, xk, emb, wqe, ek, wp, interpret=False):
    grid = ((E_PAD + B_TC - 1) // B_TC,)
    return pl.pallas_call(
        _tc_body,
        grid=grid,
        in_specs=[
            pl.BlockSpec((B_TC, 32), lambda i: (i, 0)),
            pl.BlockSpec((B_TC, 32), lambda i: (i, 0)),
            pl.BlockSpec((B_TC, 16), lambda i: (i, 0)),
            pl.BlockSpec((32, NSLOT), lambda i: (0, 0)),
            pl.BlockSpec((32, NSLOT), lambda i: (0, 0)),
            pl.BlockSpec((16, NSLOT), lambda i: (0, 0)),
        ],
        out_specs=pl.BlockSpec((B_TC // 128, 128), lambda i: (i, 0)),
        out_shape=jax.ShapeDtypeStruct((E_PAD // 128, 128), jnp.float32),
        interpret=interpret,
    )(xq, xk, emb, wqe, ek, wp)


def _sc_softmax_body(pre_hbm, idx_hbm, out_hbm, pre_v, idx_v, e_v, mloc,
                     sloc, glob, comb, tmp, xbuf, ibuf, sh_m, sh_g):
    wid = lax.axis_index("s")
    base = wid * CH
    nbase = wid * NS

    pltpu.sync_copy(pre_hbm.at[pl.ds(base, CH)], pre_v)
    pltpu.sync_copy(idx_hbm.at[pl.ds(base, CH)], idx_v)

    neg = jnp.full((L,), -1e30, jnp.float32)
    zero = jnp.zeros((L,), jnp.float32)

    def init_body(i, c):
        mloc[pl.ds(i * L, L)] = neg
        sloc[pl.ds(i * L, L)] = zero
        return c

    lax.fori_loop(0, NP // L, init_body, 0)

    ibuf[pl.ds(0, L)] = jnp.full((L,), -1, jnp.int32)
    ibuf[pl.ds(2 * L, L)] = jnp.full((L,), -2, jnp.int32)

    xbuf[pl.ds(0, L)] = neg

    def max_step(v, c):
        off = v * L
        x = pre_v[pl.ds(off, L)]
        ids = idx_v[pl.ds(off, L)]
        ibuf[pl.ds(L, L)] = ids
        for d in (1, 2, 4, 8):
            xbuf[pl.ds(L, L)] = x
            y = xbuf[pl.ds(L - d, L)]
            pid = ibuf[pl.ds(L - d, L)]
            x = jnp.where(ids == pid, jnp.maximum(x, y), x)
        nid = ibuf[pl.ds(L + 1, L)]
        endm = ids != nid
        old = plsc.load_gather(mloc, [ids])
        plsc.store_scatter(mloc, [ids], jnp.maximum(x, old), mask=endm)
        return c

    lax.fori_loop(0, VEC, max_step, 0)

    pltpu.sync_copy(mloc, sh_m.at[wid])
    plsc.subcore_barrier()
    for w2 in range(NW):
        pltpu.sync_copy(sh_m.at[w2, pl.ds(nbase, NS)], comb.at[w2])

    def red_max(j, c):
        acc = comb[0, pl.ds(j * L, L)]
        for w2 in range(1, NW):
            acc = jnp.maximum(acc, comb[w2, pl.ds(j * L, L)])
        tmp[pl.ds(j * L, L)] = acc
        return c

    lax.fori_loop(0, NS // L, red_max, 0)
    pltpu.sync_copy(tmp, sh_g.at[pl.ds(nbase, NS)])
    plsc.subcore_barrier()
    pltpu.sync_copy(sh_g, glob)

    xbuf[pl.ds(0, L)] = zero

    def sum_step(v, c):
        off = v * L
        p = pre_v[pl.ds(off, L)]
        ids = idx_v[pl.ds(off, L)]
        z = plsc.load_gather(glob, [ids])
        x = jnp.exp(p - z)
        e_v[pl.ds(off, L)] = x
        ibuf[pl.ds(L, L)] = ids
        for d in (1, 2, 4, 8):
            xbuf[pl.ds(L, L)] = x
            y = xbuf[pl.ds(L - d, L)]
            pid = ibuf[pl.ds(L - d, L)]
            x = jnp.where(ids == pid, x + y, x)
        nid = ibuf[pl.ds(L + 1, L)]
        endm = ids != nid
        old = plsc.load_gather(sloc, [ids])
        plsc.store_scatter(sloc, [ids], x + old, mask=endm)
        return c

    lax.fori_loop(0, VEC, sum_step, 0)

    pltpu.sync_copy(sloc, sh_m.at[wid])
    plsc.subcore_barrier()
    for w2 in range(NW):
        pltpu.sync_copy(sh_m.at[w2, pl.ds(nbase, NS)], comb.at[w2])

    def red_sum(j, c):
        acc = comb[0, pl.ds(j * L, L)]
        for w2 in range(1, NW):
            acc = acc + comb[w2, pl.ds(j * L, L)]
        tmp[pl.ds(j * L, L)] = acc
        return c

    lax.fori_loop(0, NS // L, red_sum, 0)
    pltpu.sync_copy(tmp, sh_g.at[pl.ds(nbase, NS)])
    plsc.subcore_barrier()
    pltpu.sync_copy(sh_g, glob)

    def out_step(v, c):
        off = v * L
        e = e_v[pl.ds(off, L)]
        ids = idx_v[pl.ds(off, L)]
        s = plsc.load_gather(glob, [ids])
        e_v[pl.ds(off, L)] = e / (s + 1e-16)
        return c

    lax.fori_loop(0, VEC, out_step, 0)
    pltpu.sync_copy(e_v, out_hbm.at[pl.ds(base, CH)])


def _sc_softmax(pre, idx):
    mesh = plsc.VectorSubcoreMesh(
        core_axis_name="c", subcore_axis_name="s", num_cores=1,
        num_subcores=NW)
    fn = pl.kernel(
        _sc_softmax_body,
        out_type=jax.ShapeDtypeStruct((E_PAD,), jnp.float32),
        mesh=mesh,
        scratch_types=[
            pltpu.VMEM((CH,), jnp.float32),
            pltpu.VMEM((CH,), jnp.int32),
            pltpu.VMEM((CH,), jnp.float32),
            pltpu.VMEM((NP,), jnp.float32),
            pltpu.VMEM((NP,), jnp.float32),
            pltpu.VMEM((NP,), jnp.float32),
            pltpu.VMEM((NW, NS), jnp.float32),
            pltpu.VMEM((NS,), jnp.float32),
            pltpu.VMEM((2 * L,), jnp.float32),
            pltpu.VMEM((3 * L,), jnp.int32),
            pltpu.VMEM_SHARED((NW, NP), jnp.float32),
            pltpu.VMEM_SHARED((NP,), jnp.float32),
        ],
        compiler_params=pltpu.CompilerParams(needs_layout_passes=False),
    )
    return fn(pre, idx)


def _build_consts():
    m = np.zeros((32 * 32, 384), np.float32)

    def put(rb, cb, w, sign):
        for c in range(8):
            for o in range(8):
                m[(rb * 8 + c) * 32 + cb * 8 + o, w * 64 + c * 8 + o] = sign

    put(0, 0, 0, 1.0)
    put(0, 2, 1, 1.0)
    put(1, 1, 4, 1.0)
    put(1, 3, 5, -1.0)
    put(2, 0, 2, 1.0)
    put(2, 2, 3, 1.0)
    put(3, 1, 5, 1.0)
    put(3, 3, 4, 1.0)

    scale = 8.0 ** -0.5
    cq = np.zeros((32, NSLOT), np.float32)
    ekc = np.zeros((32, NSLOT), np.float32)
    cp = np.zeros((384, NSLOT), np.float32)
    for s in range(8):
        for c in range(8):
            for o in range(8):
                col = s * 64 + c * 8 + o
                cq[_GAMMA[s] * 8 + o, col] = _SIGN[s] * scale
                ekc[_BETA[s] * 8 + c, col] = 1.0
                cp[_WMAP[s] * 64 + c * 8 + o, col] = 1.0
    return m, cq, ekc, cp


_M_B, _C_Q, _EK_CONST, _C_P = _build_consts()


def _prepack(Wq, Wprod):
    wbig = (jnp.asarray(_M_B) @ Wq.reshape(384)).reshape(32, 32)
    wqe = wbig @ jnp.asarray(_C_Q)
    ek = jnp.asarray(_EK_CONST)
    wp = Wprod @ jnp.asarray(_C_P)
    return wqe, ek, wp


def kernel(x_q, x_k, edge_emb, Wq, Wprod, index, num_nodes):
    e = x_q.shape[0]
    xq = x_q.reshape(e, 32)
    xk = x_k.reshape(e, 32)
    idx = jnp.minimum(index, num_nodes - 1).astype(jnp.int32)

    wqe, ek, wp = _prepack(Wq, Wprod)
    pre = _tc_pre(xq, xk, edge_emb, wqe, ek, wp)
    out = _sc_softmax(pre.reshape(E_PAD), idx)
    return out.reshape(e, 1)


# speedup vs baseline: 2.5384x; 1.5596x over previous
import functools

import numpy as np

import jax
import jax.numpy as jnp
from jax import lax
from jax.experimental import pallas as pl
from jax.experimental.pallas import tpu as pltpu
from jax.experimental.pallas import tpu_sc as plsc

E_TOTAL = 160000
N_NODES = 10000
E_PAD = 160000
B_TC = 3200
NSLOT = 512

NW = 16
L = 16
CH = E_PAD // NW
VEC = CH // L
NP = 10240
NS = NP // NW

_GAMMA = (0, 2, 0, 2, 1, 3, 1, 3)
_BETA = (0, 0, 2, 2, 1, 3, 3, 1)
_SIGN = (1.0, 1.0, 1.0, 1.0, 1.0, 1.0, 1.0, -1.0)
_WMAP = (0, 1, 2, 3, 4, 4, 5, 5)


def _tc_body(xq_ref, xk_ref, emb_ref, wqe_ref, ek_ref, wp_ref, out_ref):
    q = jnp.dot(wqe_ref[...], xq_ref[...], preferred_element_type=jnp.float32)
    k = jnp.dot(ek_ref[...], xk_ref[...], preferred_element_type=jnp.float32)
    w = jnp.dot(wp_ref[...], emb_ref[...], preferred_element_type=jnp.float32)
    p = q * k * w
    out_ref[...] = jnp.sum(p, axis=0).reshape(1, 1, B_TC)


def _tc_pre(xq, xk, emb, wqe, ek, wp, interpret=False):
    grid = (E_PAD // B_TC,)
    return pl.pallas_call(
        _tc_body,
        grid=grid,
        in_specs=[
            pl.BlockSpec((32, B_TC), lambda i: (0, i)),
            pl.BlockSpec((32, B_TC), lambda i: (0, i)),
            pl.BlockSpec((16, B_TC), lambda i: (0, i)),
            pl.BlockSpec((NSLOT, 32), lambda i: (0, 0)),
            pl.BlockSpec((NSLOT, 32), lambda i: (0, 0)),
            pl.BlockSpec((NSLOT, 16), lambda i: (0, 0)),
        ],
        out_specs=pl.BlockSpec((1, 1, B_TC), lambda i: (i, 0, 0)),
        out_shape=jax.ShapeDtypeStruct((E_PAD // B_TC, 1, B_TC), jnp.float32),
        interpret=interpret,
    )(xq, xk, emb, wqe, ek, wp)


def _sc_softmax_body(pre_hbm, idx_hbm, out_hbm, pre_v, idx_v, e_v, mloc,
                     sloc, glob, comb, tmp, xbuf, ibuf, sh_m, sh_g):
    wid = lax.axis_index("s")
    base = wid * CH
    nbase = wid * NS

    pltpu.sync_copy(pre_hbm.at[pl.ds(base, CH)], pre_v)
    pltpu.sync_copy(idx_hbm.at[pl.ds(base, CH)], idx_v)

    neg = jnp.full((L,), -1e30, jnp.float32)
    zero = jnp.zeros((L,), jnp.float32)

    def init_body(i, c):
        mloc[pl.ds(i * L, L)] = neg
        sloc[pl.ds(i * L, L)] = zero
        return c

    lax.fori_loop(0, NP // L, init_body, 0)

    ibuf[pl.ds(0, L)] = jnp.full((L,), -1, jnp.int32)
    ibuf[pl.ds(2 * L, L)] = jnp.full((L,), -2, jnp.int32)

    xbuf[pl.ds(0, L)] = neg

    def max_step(v, c):
        off = v * L
        x = pre_v[pl.ds(off, L)]
        ids = idx_v[pl.ds(off, L)]
        ibuf[pl.ds(L, L)] = ids
        for d in (1, 2, 4, 8):
            xbuf[pl.ds(L, L)] = x
            y = xbuf[pl.ds(L - d, L)]
            pid = ibuf[pl.ds(L - d, L)]
            x = jnp.where(ids == pid, jnp.maximum(x, y), x)
        nid = ibuf[pl.ds(L + 1, L)]
        endm = ids != nid
        old = plsc.load_gather(mloc, [ids])
        plsc.store_scatter(mloc, [ids], jnp.maximum(x, old), mask=endm)
        return c

    lax.fori_loop(0, VEC, max_step, 0)

    pltpu.sync_copy(mloc, sh_m.at[wid])
    plsc.subcore_barrier()
    for w2 in range(NW):
        pltpu.sync_copy(sh_m.at[w2, pl.ds(nbase, NS)], comb.at[w2])

    def red_max(j, c):
        acc = comb[0, pl.ds(j * L, L)]
        for w2 in range(1, NW):
            acc = jnp.maximum(acc, comb[w2, pl.ds(j * L, L)])
        tmp[pl.ds(j * L, L)] = acc
        return c

    lax.fori_loop(0, NS // L, red_max, 0)
    pltpu.sync_copy(tmp, sh_g.at[pl.ds(nbase, NS)])
    plsc.subcore_barrier()
    pltpu.sync_copy(sh_g, glob)

    xbuf[pl.ds(0, L)] = zero

    def sum_step(v, c):
        off = v * L
        p = pre_v[pl.ds(off, L)]
        ids = idx_v[pl.ds(off, L)]
        z = plsc.load_gather(glob, [ids])
        x = jnp.exp(p - z)
        e_v[pl.ds(off, L)] = x
        ibuf[pl.ds(L, L)] = ids
        for d in (1, 2, 4, 8):
            xbuf[pl.ds(L, L)] = x
            y = xbuf[pl.ds(L - d, L)]
            pid = ibuf[pl.ds(L - d, L)]
            x = jnp.where(ids == pid, x + y, x)
        nid = ibuf[pl.ds(L + 1, L)]
        endm = ids != nid
        old = plsc.load_gather(sloc, [ids])
        plsc.store_scatter(sloc, [ids], x + old, mask=endm)
        return c

    lax.fori_loop(0, VEC, sum_step, 0)

    pltpu.sync_copy(sloc, sh_m.at[wid])
    plsc.subcore_barrier()
    for w2 in range(NW):
        pltpu.sync_copy(sh_m.at[w2, pl.ds(nbase, NS)], comb.at[w2])

    def red_sum(j, c):
        acc = comb[0, pl.ds(j * L, L)]
        for w2 in range(1, NW):
            acc = acc + comb[w2, pl.ds(j * L, L)]
        tmp[pl.ds(j * L, L)] = acc
        return c

    lax.fori_loop(0, NS // L, red_sum, 0)
    pltpu.sync_copy(tmp, sh_g.at[pl.ds(nbase, NS)])
    plsc.subcore_barrier()
    pltpu.sync_copy(sh_g, glob)

    def out_step(v, c):
        off = v * L
        e = e_v[pl.ds(off, L)]
        ids = idx_v[pl.ds(off, L)]
        s = plsc.load_gather(glob, [ids])
        e_v[pl.ds(off, L)] = e / (s + 1e-16)
        return c

    lax.fori_loop(0, VEC, out_step, 0)
    pltpu.sync_copy(e_v, out_hbm.at[pl.ds(base, CH)])


def _sc_softmax(pre, idx):
    mesh = plsc.VectorSubcoreMesh(
        core_axis_name="c", subcore_axis_name="s", num_cores=1,
        num_subcores=NW)
    fn = pl.kernel(
        _sc_softmax_body,
        out_type=jax.ShapeDtypeStruct((E_PAD,), jnp.float32),
        mesh=mesh,
        scratch_types=[
            pltpu.VMEM((CH,), jnp.float32),
            pltpu.VMEM((CH,), jnp.int32),
            pltpu.VMEM((CH,), jnp.float32),
            pltpu.VMEM((NP,), jnp.float32),
            pltpu.VMEM((NP,), jnp.float32),
            pltpu.VMEM((NP,), jnp.float32),
            pltpu.VMEM((NW, NS), jnp.float32),
            pltpu.VMEM((NS,), jnp.float32),
            pltpu.VMEM((2 * L,), jnp.float32),
            pltpu.VMEM((3 * L,), jnp.int32),
            pltpu.VMEM_SHARED((NW, NP), jnp.float32),
            pltpu.VMEM_SHARED((NP,), jnp.float32),
        ],
        compiler_params=pltpu.CompilerParams(needs_layout_passes=False),
    )
    return fn(pre, idx)


def _build_consts():
    m = np.zeros((32 * 32, 384), np.float32)

    def put(rb, cb, w, sign):
        for c in range(8):
            for o in range(8):
                m[(rb * 8 + c) * 32 + cb * 8 + o, w * 64 + c * 8 + o] = sign

    put(0, 0, 0, 1.0)
    put(0, 2, 1, 1.0)
    put(1, 1, 4, 1.0)
    put(1, 3, 5, -1.0)
    put(2, 0, 2, 1.0)
    put(2, 2, 3, 1.0)
    put(3, 1, 5, 1.0)
    put(3, 3, 4, 1.0)

    scale = 8.0 ** -0.5
    cq = np.zeros((32, NSLOT), np.float32)
    ekc = np.zeros((32, NSLOT), np.float32)
    cp = np.zeros((384, NSLOT), np.float32)
    for s in range(8):
        for c in range(8):
            for o in range(8):
                col = s * 64 + c * 8 + o
                cq[_GAMMA[s] * 8 + o, col] = _SIGN[s] * scale
                ekc[_BETA[s] * 8 + c, col] = 1.0
                cp[_WMAP[s] * 64 + c * 8 + o, col] = 1.0
    return m, cq, ekc, cp


_M_B, _C_Q, _EK_CONST, _C_P = _build_consts()


def _prepack(Wq, Wprod):
    hi = jax.lax.Precision.HIGHEST
    wbig = jnp.dot(jnp.asarray(_M_B), Wq.reshape(384), precision=hi).reshape(32, 32)
    wqe_t = jnp.dot(wbig, jnp.asarray(_C_Q), precision=hi).T
    ek_t = jnp.asarray(_EK_CONST.T)
    wp_t = jnp.dot(Wprod, jnp.asarray(_C_P), precision=hi).T
    return wqe_t, ek_t, wp_t


def kernel(x_q, x_k, edge_emb, Wq, Wprod, index, num_nodes):
    e = x_q.shape[0]
    xq = x_q.reshape(e, 32).T
    xk = x_k.reshape(e, 32).T
    emb = edge_emb.T
    idx = jnp.minimum(index, num_nodes - 1).astype(jnp.int32)

    wqe, ek, wp = _prepack(Wq, Wprod)
    pre = _tc_pre(xq, xk, emb, wqe, ek, wp)
    out = _sc_softmax(pre.reshape(E_PAD), idx)
    return out.reshape(e, 1)


# speedup vs baseline: 2.5896x; 1.0202x over previous
import functools

import numpy as np

import jax
import jax.numpy as jnp
from jax import lax
from jax.experimental import pallas as pl
from jax.experimental.pallas import tpu as pltpu
from jax.experimental.pallas import tpu_sc as plsc

E_TOTAL = 160000
N_NODES = 10000
E_PAD = 160000
B_TC = 6400
NSLOT = 512

NW = 16
L = 16
CH = E_PAD // NW
VEC = CH // L
NP = 10240
NS = NP // NW

_GAMMA = (0, 2, 0, 2, 1, 3, 1, 3)
_BETA = (0, 0, 2, 2, 1, 3, 3, 1)
_SIGN = (1.0, 1.0, 1.0, 1.0, 1.0, 1.0, 1.0, -1.0)
_WMAP = (0, 1, 2, 3, 4, 4, 5, 5)


def _tc_body(xq_ref, xk_ref, emb_ref, wqe_ref, ek_ref, wp_ref, out_ref):
    q = jnp.dot(wqe_ref[...], xq_ref[...], preferred_element_type=jnp.float32)
    k = jnp.dot(ek_ref[...], xk_ref[...], preferred_element_type=jnp.float32)
    w = jnp.dot(wp_ref[...], emb_ref[...], preferred_element_type=jnp.float32)
    p = q * k * w
    out_ref[...] = jnp.sum(p, axis=0).reshape(1, 1, B_TC)


def _tc_pre(xq, xk, emb, wqe, ek, wp, interpret=False):
    grid = (E_PAD // B_TC,)
    return pl.pallas_call(
        _tc_body,
        grid=grid,
        in_specs=[
            pl.BlockSpec((32, B_TC), lambda i: (0, i)),
            pl.BlockSpec((32, B_TC), lambda i: (0, i)),
            pl.BlockSpec((16, B_TC), lambda i: (0, i)),
            pl.BlockSpec((NSLOT, 32), lambda i: (0, 0)),
            pl.BlockSpec((NSLOT, 32), lambda i: (0, 0)),
            pl.BlockSpec((NSLOT, 16), lambda i: (0, 0)),
        ],
        out_specs=pl.BlockSpec((1, 1, B_TC), lambda i: (i, 0, 0)),
        out_shape=jax.ShapeDtypeStruct((E_PAD // B_TC, 1, B_TC), jnp.float32),
        interpret=interpret,
    )(xq, xk, emb, wqe, ek, wp)


def _sc_softmax_body(pre_hbm, idx_hbm, out_hbm, pre_v, idx_v, e_v, mloc,
                     sloc, glob, comb, tmp, xbuf, ibuf, sh_m, sh_g):
    wid = lax.axis_index("s")
    base = wid * CH
    nbase = wid * NS

    pltpu.sync_copy(pre_hbm.at[pl.ds(base, CH)], pre_v)
    pltpu.sync_copy(idx_hbm.at[pl.ds(base, CH)], idx_v)

    neg = jnp.full((L,), -1e30, jnp.float32)
    zero = jnp.zeros((L,), jnp.float32)

    def init_body(i, c):
        mloc[pl.ds(i * L, L)] = neg
        sloc[pl.ds(i * L, L)] = zero
        return c

    lax.fori_loop(0, NP // L, init_body, 0)

    ibuf[pl.ds(0, L)] = jnp.full((L,), -1, jnp.int32)
    ibuf[pl.ds(2 * L, L)] = jnp.full((L,), -2, jnp.int32)

    xbuf[pl.ds(0, L)] = neg

    def max_step(v, c):
        off = v * L
        x = pre_v[pl.ds(off, L)]
        ids = idx_v[pl.ds(off, L)]
        ibuf[pl.ds(L, L)] = ids
        for d in (1, 2, 4, 8):
            xbuf[pl.ds(L, L)] = x
            y = xbuf[pl.ds(L - d, L)]
            pid = ibuf[pl.ds(L - d, L)]
            x = jnp.where(ids == pid, jnp.maximum(x, y), x)
        nid = ibuf[pl.ds(L + 1, L)]
        endm = ids != nid
        old = plsc.load_gather(mloc, [ids])
        plsc.store_scatter(mloc, [ids], jnp.maximum(x, old), mask=endm)
        return c

    lax.fori_loop(0, VEC, max_step, 0)

    pltpu.sync_copy(mloc, sh_m.at[wid])
    plsc.subcore_barrier()
    for w2 in range(NW):
        pltpu.sync_copy(sh_m.at[w2, pl.ds(nbase, NS)], comb.at[w2])

    def red_max(j, c):
        acc = comb[0, pl.ds(j * L, L)]
        for w2 in range(1, NW):
            acc = jnp.maximum(acc, comb[w2, pl.ds(j * L, L)])
        tmp[pl.ds(j * L, L)] = acc
        return c

    lax.fori_loop(0, NS // L, red_max, 0)
    pltpu.sync_copy(tmp, sh_g.at[pl.ds(nbase, NS)])
    plsc.subcore_barrier()
    pltpu.sync_copy(sh_g, glob)

    xbuf[pl.ds(0, L)] = zero

    def sum_step(v, c):
        off = v * L
        p = pre_v[pl.ds(off, L)]
        ids = idx_v[pl.ds(off, L)]
        z = plsc.load_gather(glob, [ids])
        x = jnp.exp(p - z)
        e_v[pl.ds(off, L)] = x
        ibuf[pl.ds(L, L)] = ids
        for d in (1, 2, 4, 8):
            xbuf[pl.ds(L, L)] = x
            y = xbuf[pl.ds(L - d, L)]
            pid = ibuf[pl.ds(L - d, L)]
            x = jnp.where(ids == pid, x + y, x)
        nid = ibuf[pl.ds(L + 1, L)]
        endm = ids != nid
        old = plsc.load_gather(sloc, [ids])
        plsc.store_scatter(sloc, [ids], x + old, mask=endm)
        return c

    lax.fori_loop(0, VEC, sum_step, 0)

    pltpu.sync_copy(sloc, sh_m.at[wid])
    plsc.subcore_barrier()
    for w2 in range(NW):
        pltpu.sync_copy(sh_m.at[w2, pl.ds(nbase, NS)], comb.at[w2])

    def red_sum(j, c):
        acc = comb[0, pl.ds(j * L, L)]
        for w2 in range(1, NW):
            acc = acc + comb[w2, pl.ds(j * L, L)]
        tmp[pl.ds(j * L, L)] = acc
        return c

    lax.fori_loop(0, NS // L, red_sum, 0)
    pltpu.sync_copy(tmp, sh_g.at[pl.ds(nbase, NS)])
    plsc.subcore_barrier()
    pltpu.sync_copy(sh_g, glob)

    def out_step(v, c):
        off = v * L
        e = e_v[pl.ds(off, L)]
        ids = idx_v[pl.ds(off, L)]
        s = plsc.load_gather(glob, [ids])
        e_v[pl.ds(off, L)] = e / (s + 1e-16)
        return c

    lax.fori_loop(0, VEC, out_step, 0)
    pltpu.sync_copy(e_v, out_hbm.at[pl.ds(base, CH)])


def _sc_softmax(pre, idx):
    mesh = plsc.VectorSubcoreMesh(
        core_axis_name="c", subcore_axis_name="s", num_cores=1,
        num_subcores=NW)
    fn = pl.kernel(
        _sc_softmax_body,
        out_type=jax.ShapeDtypeStruct((E_PAD,), jnp.float32),
        mesh=mesh,
        scratch_types=[
            pltpu.VMEM((CH,), jnp.float32),
            pltpu.VMEM((CH,), jnp.int32),
            pltpu.VMEM((CH,), jnp.float32),
            pltpu.VMEM((NP,), jnp.float32),
            pltpu.VMEM((NP,), jnp.float32),
            pltpu.VMEM((NP,), jnp.float32),
            pltpu.VMEM((NW, NS), jnp.float32),
            pltpu.VMEM((NS,), jnp.float32),
            pltpu.VMEM((2 * L,), jnp.float32),
            pltpu.VMEM((3 * L,), jnp.int32),
            pltpu.VMEM_SHARED((NW, NP), jnp.float32),
            pltpu.VMEM_SHARED((NP,), jnp.float32),
        ],
        compiler_params=pltpu.CompilerParams(needs_layout_passes=False),
    )
    return fn(pre, idx)


def _build_consts():
    m = np.zeros((32 * 32, 384), np.float32)

    def put(rb, cb, w, sign):
        for c in range(8):
            for o in range(8):
                m[(rb * 8 + c) * 32 + cb * 8 + o, w * 64 + c * 8 + o] = sign

    put(0, 0, 0, 1.0)
    put(0, 2, 1, 1.0)
    put(1, 1, 4, 1.0)
    put(1, 3, 5, -1.0)
    put(2, 0, 2, 1.0)
    put(2, 2, 3, 1.0)
    put(3, 1, 5, 1.0)
    put(3, 3, 4, 1.0)

    scale = 8.0 ** -0.5
    cq = np.zeros((32, NSLOT), np.float32)
    ekc = np.zeros((32, NSLOT), np.float32)
    cp = np.zeros((384, NSLOT), np.float32)
    for s in range(8):
        for c in range(8):
            for o in range(8):
                col = s * 64 + c * 8 + o
                cq[_GAMMA[s] * 8 + o, col] = _SIGN[s] * scale
                ekc[_BETA[s] * 8 + c, col] = 1.0
                cp[_WMAP[s] * 64 + c * 8 + o, col] = 1.0
    return m, cq, ekc, cp


_M_B, _C_Q, _EK_CONST, _C_P = _build_consts()


def _prepack(Wq, Wprod):
    hi = jax.lax.Precision.HIGHEST
    wbig = jnp.dot(jnp.asarray(_M_B), Wq.reshape(384), precision=hi).reshape(32, 32)
    wqe_t = jnp.dot(wbig, jnp.asarray(_C_Q), precision=hi).T
    ek_t = jnp.asarray(_EK_CONST.T)
    wp_t = jnp.dot(Wprod, jnp.asarray(_C_P), precision=hi).T
    return wqe_t, ek_t, wp_t


def kernel(x_q, x_k, edge_emb, Wq, Wprod, index, num_nodes):
    e = x_q.shape[0]
    xq = x_q.reshape(e, 32).T
    xk = x_k.reshape(e, 32).T
    emb = edge_emb.T
    idx = jnp.minimum(index, num_nodes - 1).astype(jnp.int32)

    wqe, ek, wp = _prepack(Wq, Wprod)
    pre = _tc_pre(xq, xk, emb, wqe, ek, wp)
    out = _sc_softmax(pre.reshape(E_PAD), idx)
    return out.reshape(e, 1)


# speedup vs baseline: 2.6259x; 1.0140x over previous
import functools

import numpy as np

import jax
import jax.numpy as jnp
from jax import lax
from jax.experimental import pallas as pl
from jax.experimental.pallas import tpu as pltpu
from jax.experimental.pallas import tpu_sc as plsc

E_TOTAL = 160000
N_NODES = 10000
E_PAD = 160000
B_TC = 6400
NSLOT = 512

NW = 16
L = 16
CH = E_PAD // NW
VEC = CH // L
NP = 10240
NS = NP // NW

_GAMMA = (0, 2, 0, 2, 1, 3, 1, 3)
_BETA = (0, 0, 2, 2, 1, 3, 3, 1)
_SIGN = (1.0, 1.0, 1.0, 1.0, 1.0, 1.0, 1.0, -1.0)
_WMAP = (0, 1, 2, 3, 4, 4, 5, 5)


def _tc_body(xq_ref, xk_ref, emb_ref, wqe_ref, ek_ref, wp_ref, out_ref):
    q = jnp.dot(wqe_ref[...], xq_ref[...], preferred_element_type=jnp.float32)
    k = jnp.dot(ek_ref[...], xk_ref[...], preferred_element_type=jnp.float32)
    w = jnp.dot(wp_ref[...], emb_ref[...], preferred_element_type=jnp.float32)
    p = q * k * w
    out_ref[...] = jnp.sum(p, axis=0).reshape(1, 1, B_TC)


def _tc_pre(xq, xk, emb, wqe, ek, wp, interpret=False):
    grid = (E_PAD // B_TC,)
    return pl.pallas_call(
        _tc_body,
        grid=grid,
        in_specs=[
            pl.BlockSpec((32, B_TC), lambda i: (0, i)),
            pl.BlockSpec((32, B_TC), lambda i: (0, i)),
            pl.BlockSpec((16, B_TC), lambda i: (0, i)),
            pl.BlockSpec((NSLOT, 32), lambda i: (0, 0)),
            pl.BlockSpec((NSLOT, 32), lambda i: (0, 0)),
            pl.BlockSpec((NSLOT, 16), lambda i: (0, 0)),
        ],
        out_specs=pl.BlockSpec((1, 1, B_TC), lambda i: (i, 0, 0)),
        out_shape=jax.ShapeDtypeStruct((E_PAD // B_TC, 1, B_TC), jnp.float32),
        interpret=interpret,
    )(xq, xk, emb, wqe, ek, wp)


def _sc_softmax_body(pre_hbm, idx_hbm, out_hbm, pre_v, idx_v, e_v, mloc,
                     sloc, glob, comb, tmp, xbuf, ibuf, sh_m, sh_g):
    wid = lax.axis_index("s")
    base = wid * CH
    nbase = wid * NS

    pltpu.sync_copy(pre_hbm.at[pl.ds(base, CH)], pre_v)
    pltpu.sync_copy(idx_hbm.at[pl.ds(base, CH)], idx_v)

    neg = jnp.full((L,), -1e30, jnp.float32)
    zero = jnp.zeros((L,), jnp.float32)

    def init_body(i, c):
        mloc[pl.ds(i * L, L)] = neg
        sloc[pl.ds(i * L, L)] = zero
        return c

    lax.fori_loop(0, NP // L, init_body, 0)

    ibuf[pl.ds(0, L)] = jnp.full((L,), -1, jnp.int32)
    ibuf[pl.ds(2 * L, L)] = jnp.full((L,), -2, jnp.int32)

    xbuf[pl.ds(0, L)] = neg

    def max_step(v, c):
        off = v * L
        x = pre_v[pl.ds(off, L)]
        ids = idx_v[pl.ds(off, L)]
        ibuf[pl.ds(L, L)] = ids
        for d in (1, 2, 4, 8):
            xbuf[pl.ds(L, L)] = x
            y = xbuf[pl.ds(L - d, L)]
            pid = ibuf[pl.ds(L - d, L)]
            x = jnp.where(ids == pid, jnp.maximum(x, y), x)
        nid = ibuf[pl.ds(L + 1, L)]
        endm = ids != nid
        old = plsc.load_gather(mloc, [ids])
        plsc.store_scatter(mloc, [ids], jnp.maximum(x, old), mask=endm)
        return c

    lax.fori_loop(0, VEC, max_step, 0)

    pltpu.sync_copy(mloc, sh_m.at[wid])
    plsc.subcore_barrier()
    pltpu.sync_copy(sh_m.at[:, pl.ds(nbase, NS)], comb)

    def red_max(j, c):
        acc = comb[0, pl.ds(j * L, L)]
        for w2 in range(1, NW):
            acc = jnp.maximum(acc, comb[w2, pl.ds(j * L, L)])
        tmp[pl.ds(j * L, L)] = acc
        return c

    lax.fori_loop(0, NS // L, red_max, 0)
    pltpu.sync_copy(tmp, sh_g.at[pl.ds(nbase, NS)])
    plsc.subcore_barrier()
    pltpu.sync_copy(sh_g, glob)

    xbuf[pl.ds(0, L)] = zero

    def sum_step(v, c):
        off = v * L
        p = pre_v[pl.ds(off, L)]
        ids = idx_v[pl.ds(off, L)]
        z = plsc.load_gather(glob, [ids])
        x = jnp.exp(p - z)
        e_v[pl.ds(off, L)] = x
        ibuf[pl.ds(L, L)] = ids
        for d in (1, 2, 4, 8):
            xbuf[pl.ds(L, L)] = x
            y = xbuf[pl.ds(L - d, L)]
            pid = ibuf[pl.ds(L - d, L)]
            x = jnp.where(ids == pid, x + y, x)
        nid = ibuf[pl.ds(L + 1, L)]
        endm = ids != nid
        old = plsc.load_gather(sloc, [ids])
        plsc.store_scatter(sloc, [ids], x + old, mask=endm)
        return c

    lax.fori_loop(0, VEC, sum_step, 0)

    pltpu.sync_copy(sloc, sh_m.at[wid])
    plsc.subcore_barrier()
    pltpu.sync_copy(sh_m.at[:, pl.ds(nbase, NS)], comb)

    def red_sum(j, c):
        acc = comb[0, pl.ds(j * L, L)]
        for w2 in range(1, NW):
            acc = acc + comb[w2, pl.ds(j * L, L)]
        tmp[pl.ds(j * L, L)] = acc
        return c

    lax.fori_loop(0, NS // L, red_sum, 0)
    pltpu.sync_copy(tmp, sh_g.at[pl.ds(nbase, NS)])
    plsc.subcore_barrier()
    pltpu.sync_copy(sh_g, glob)

    def out_step(v, c):
        off = v * L
        e = e_v[pl.ds(off, L)]
        ids = idx_v[pl.ds(off, L)]
        s = plsc.load_gather(glob, [ids])
        e_v[pl.ds(off, L)] = e / (s + 1e-16)
        return c

    lax.fori_loop(0, VEC, out_step, 0)
    pltpu.sync_copy(e_v, out_hbm.at[pl.ds(base, CH)])


def _sc_softmax(pre, idx):
    mesh = plsc.VectorSubcoreMesh(
        core_axis_name="c", subcore_axis_name="s", num_cores=1,
        num_subcores=NW)
    fn = pl.kernel(
        _sc_softmax_body,
        out_type=jax.ShapeDtypeStruct((E_PAD,), jnp.float32),
        mesh=mesh,
        scratch_types=[
            pltpu.VMEM((CH,), jnp.float32),
            pltpu.VMEM((CH,), jnp.int32),
            pltpu.VMEM((CH,), jnp.float32),
            pltpu.VMEM((NP,), jnp.float32),
            pltpu.VMEM((NP,), jnp.float32),
            pltpu.VMEM((NP,), jnp.float32),
            pltpu.VMEM((NW, NS), jnp.float32),
            pltpu.VMEM((NS,), jnp.float32),
            pltpu.VMEM((2 * L,), jnp.float32),
            pltpu.VMEM((3 * L,), jnp.int32),
            pltpu.VMEM_SHARED((NW, NP), jnp.float32),
            pltpu.VMEM_SHARED((NP,), jnp.float32),
        ],
        compiler_params=pltpu.CompilerParams(needs_layout_passes=False),
    )
    return fn(pre, idx)


def _build_consts():
    m = np.zeros((32 * 32, 384), np.float32)

    def put(rb, cb, w, sign):
        for c in range(8):
            for o in range(8):
                m[(rb * 8 + c) * 32 + cb * 8 + o, w * 64 + c * 8 + o] = sign

    put(0, 0, 0, 1.0)
    put(0, 2, 1, 1.0)
    put(1, 1, 4, 1.0)
    put(1, 3, 5, -1.0)
    put(2, 0, 2, 1.0)
    put(2, 2, 3, 1.0)
    put(3, 1, 5, 1.0)
    put(3, 3, 4, 1.0)

    scale = 8.0 ** -0.5
    cq = np.zeros((32, NSLOT), np.float32)
    ekc = np.zeros((32, NSLOT), np.float32)
    cp = np.zeros((384, NSLOT), np.float32)
    for s in range(8):
        for c in range(8):
            for o in range(8):
                col = s * 64 + c * 8 + o
                cq[_GAMMA[s] * 8 + o, col] = _SIGN[s] * scale
                ekc[_BETA[s] * 8 + c, col] = 1.0
                cp[_WMAP[s] * 64 + c * 8 + o, col] = 1.0
    return m, cq, ekc, cp


_M_B, _C_Q, _EK_CONST, _C_P = _build_consts()


def _prepack(Wq, Wprod):
    hi = jax.lax.Precision.HIGHEST
    wbig = jnp.dot(jnp.asarray(_M_B), Wq.reshape(384), precision=hi).reshape(32, 32)
    wqe_t = jnp.dot(wbig, jnp.asarray(_C_Q), precision=hi).T
    ek_t = jnp.asarray(_EK_CONST.T)
    wp_t = jnp.dot(Wprod, jnp.asarray(_C_P), precision=hi).T
    return wqe_t, ek_t, wp_t


def kernel(x_q, x_k, edge_emb, Wq, Wprod, index, num_nodes):
    e = x_q.shape[0]
    xq = x_q.reshape(e, 32).T
    xk = x_k.reshape(e, 32).T
    emb = edge_emb.T
    idx = jnp.minimum(index, num_nodes - 1).astype(jnp.int32)

    wqe, ek, wp = _prepack(Wq, Wprod)
    pre = _tc_pre(xq, xk, emb, wqe, ek, wp)
    out = _sc_softmax(pre.reshape(E_PAD), idx)
    return out.reshape(e, 1)
